# baseline (speedup 1.0000x reference)
import functools

import jax
import jax.numpy as jnp
from jax import lax
from jax.experimental import pallas as pl
from jax.experimental.pallas import tpu as pltpu
from jax.experimental.pallas import tpu_sc as plsc

N = 10000
E = 320000
D = 128
DH = D // 2
NC, NS = 2, 16
EPS = E // NS
CH = 80
NCH = EPS // CH
NPAD = 10240
STRIPE = NPAD // NS
FB = 80
NBF = STRIPE // FB
NBL = (N - (NS - 1) * STRIPE) // FB

_mesh = plsc.VectorSubcoreMesh(core_axis_name="c", subcore_axis_name="s")


def _rsqrt_newton(x):
    i = plsc.bitcast(x, jnp.int32)
    seed = jnp.full((16,), 0x5F3759DF, jnp.int32)
    y = plsc.bitcast(seed - lax.shift_right_logical(i, 1), jnp.float32)
    half = x * 0.5
    for _ in range(3):
        y = y * (1.5 - half * y * y)
    return y


@functools.partial(
    pl.kernel,
    out_type=(
        jax.ShapeDtypeStruct((NC, NPAD, DH), jnp.float32),
        jax.ShapeDtypeStruct((NPAD,), jnp.float32),
    ),
    mesh=_mesh,
    scratch_types=[
        pltpu.VMEM((NCH, CH), jnp.int32),
        pltpu.VMEM((NCH, CH), jnp.int32),
        pltpu.VMEM((4, CH, DH), jnp.float32),
        pltpu.VMEM((NPAD // 128, 128), jnp.float32),
        pltpu.VMEM((NPAD // 128, 128), jnp.float32),
        pltpu.VMEM((4, 2, 5, 128), jnp.float32),
        pltpu.VMEM((STRIPE,), jnp.float32),
        pltpu.VMEM((STRIPE,), jnp.float32),
        pltpu.VMEM_SHARED((NPAD, DH), jnp.float32),
        pltpu.HBM((NC, NS, 2, NPAD // 128, 128), jnp.float32),
        pltpu.HBM((NC, N, DH), jnp.float32),
        [pltpu.SemaphoreType.DMA] * 4,
        [pltpu.SemaphoreType.DMA] * 4,
    ],
    compiler_params=pltpu.CompilerParams(use_tc_tiling_on_sc=False,
                                         needs_layout_passes=False),
)
def _gcn_kernel(feat3_hbm, edges_hbm, zrows_hbm, out_hbm, normi_hbm,
                idx_s, idx_d, rows, hsrc, hdst, hstg, m0, m1,
                h_sh, hx_scr, fs_scr, sem_g, sem_s):
    c = lax.axis_index("c")
    s = lax.axis_index("s")
    r0 = s * STRIPE

    pltpu.sync_copy(zrows_hbm, h_sh.at[pl.ds(r0, STRIPE)])
    pltpu.sync_copy(edges_hbm.at[0, s], idx_s)
    pltpu.sync_copy(edges_hbm.at[1, s], idx_d)

    z16 = jnp.zeros((16,), jnp.float32)

    def zero_body2(i, carry):
        r = i // 8
        sl = pl.ds((i % 8) * 16, 16)
        hsrc[r, sl] = z16
        hdst[r, sl] = z16
        return carry

    lax.fori_loop(0, NPAD // 16, zero_body2, 0)

    ones16 = jnp.ones((16,), jnp.float32)
    c127 = jnp.full((16,), 127, jnp.int32)

    def hist_body(g, carry):
        for k in range(CH // 16):
            vs = idx_s[g, pl.ds(k * 16, 16)]
            plsc.addupdate_scatter(
                hsrc, [lax.shift_right_logical(vs, 7), vs & c127], ones16)
            vd = idx_d[g, pl.ds(k * 16, 16)]
            plsc.addupdate_scatter(
                hdst, [lax.shift_right_logical(vd, 7), vd & c127], ones16)
        return carry

    lax.fori_loop(0, NCH, hist_body, 0)

    pltpu.sync_copy(hsrc, hx_scr.at[c, s, 0])
    pltpu.sync_copy(hdst, hx_scr.at[c, s, 1])
    plsc.subcore_barrier()

    for t4 in range(NS // 4):
        pltpu.sync_copy(
            hx_scr.at[c, pl.ds(4 * t4, 4), :, pl.ds(5 * s, 5)], hstg)

        def merge_body(i, carry):
            r = i // 8
            sl = pl.ds((i % 8) * 16, 16)
            a0 = hstg[0, 0, r, sl] + hstg[1, 0, r, sl]
            a0 = a0 + hstg[2, 0, r, sl] + hstg[3, 0, r, sl]
            a1 = hstg[0, 1, r, sl] + hstg[1, 1, r, sl]
            a1 = a1 + hstg[2, 1, r, sl] + hstg[3, 1, r, sl]
            osl = pl.ds(i * 16, 16)
            if t4 == 0:
                m0[osl] = a0
                m1[osl] = a1
            else:
                m0[osl] = m0[osl] + a0
                m1[osl] = m1[osl] + a1
            return carry

        lax.fori_loop(0, STRIPE // 16, merge_body, 0)

    def norm_body(i, carry):
        sl = pl.ds(i * 16, 16)
        m0[sl] = _rsqrt_newton(jnp.maximum(m0[sl], 1.0) + 1.0)
        m1[sl] = _rsqrt_newton(jnp.maximum(m1[sl], 1.0) + 1.0)
        return carry

    lax.fori_loop(0, STRIPE // 16, norm_body, 0)

    @pl.when(c == 0)
    def _():
        pltpu.sync_copy(m1, normi_hbm.at[pl.ds(r0, STRIPE)])

    def scale_block(b):
        rb = r0 + b * FB
        pltpu.sync_copy(feat3_hbm.at[pl.ds(rb, FB), c], rows.at[0])

        def grp(j, carry):
            nv = m0[pl.ds(b * FB + 16 * j, 16)]
            for t in range(16):
                sc = nv[t]
                for k in range(DH // 16):
                    rows[0, 16 * j + t, pl.ds(k * 16, 16)] = (
                        rows[0, 16 * j + t, pl.ds(k * 16, 16)] * sc)
            return carry

        lax.fori_loop(0, FB // 16, grp, 0)
        pltpu.sync_copy(rows.at[0], fs_scr.at[c, pl.ds(rb, FB)])

    @pl.when(s < NS - 1)
    def _():
        for b in range(NBF):
            scale_block(b)

    @pl.when(s == NS - 1)
    def _():
        for b in range(NBL):
            scale_block(b)

    plsc.subcore_barrier()

    fsrc = fs_scr.at[c]

    def gather(g, b):
        pltpu.async_copy(fsrc.at[idx_s.at[g]], rows.at[b], sem_g[b])

    def wait_gather(g, b):
        pltpu.make_async_copy(fsrc.at[idx_s.at[g]], rows.at[b], sem_g[b]).wait()

    def scatter(g, b):
        pltpu.async_copy(rows.at[b], h_sh.at[idx_d.at[g]], sem_s[b], add=True)

    def wait_scatter(g, b):
        pltpu.make_async_copy(rows.at[b], h_sh.at[idx_d.at[g]],
                              sem_s[b]).wait()

    def step(g, b_main, b_pre, skip_ws=False, skip_gather=False):
        if not skip_ws:
            wait_scatter(g - 2, b_pre)
        if not skip_gather:
            gather(g + 2, b_pre)
        wait_gather(g, b_main)
        scatter(g, b_main)

    gather(0, 0)
    gather(1, 1)
    step(0, 0, 2, skip_ws=True)
    step(1, 1, 3, skip_ws=True)

    def pipe_body(gg, carry):
        g = 4 * gg + 2
        for k in range(4):
            step(g + k, (2 + k) % 4, k % 4)
        return carry

    nsteady = (NCH - 4) // 4
    lax.fori_loop(0, nsteady, pipe_body, 0)
    for g in range(2 + 4 * nsteady, NCH - 2):
        step(g, g % 4, (g + 2) % 4)
    for g in range(NCH - 2, NCH):
        step(g, g % 4, (g + 2) % 4, skip_gather=True)
    wait_scatter(NCH - 2, (NCH - 2) % 4)
    wait_scatter(NCH - 1, (NCH - 1) % 4)

    plsc.subcore_barrier()
    pltpu.sync_copy(h_sh.at[pl.ds(r0, STRIPE)],
                    out_hbm.at[c, pl.ds(r0, STRIPE)])


_RB = 2000


def _final_body(feat_ref, h_ref, nrm_ref, out_ref):
    h = jnp.concatenate([h_ref[0], h_ref[1]], axis=1)
    out_ref[...] = (feat_ref[...] + h) * nrm_ref[...]


def _final(feat, h2, normi):
    return pl.pallas_call(
        _final_body,
        grid=(N // _RB,),
        in_specs=[
            pl.BlockSpec((_RB, D), lambda i: (i, 0)),
            pl.BlockSpec((NC, _RB, DH), lambda i: (0, i, 0)),
            pl.BlockSpec((_RB, 1), lambda i: (i, 0)),
        ],
        out_specs=pl.BlockSpec((_RB, D), lambda i: (i, 0)),
        out_shape=jax.ShapeDtypeStruct((N, D), jnp.float32),
    )(feat, h2, normi)


def kernel(feat, edge_index):
    ei = edge_index.astype(jnp.int32)
    edges = ei.reshape(2, NS, NCH, CH)
    feat3 = feat.reshape(N, NC, DH)
    zrows = jnp.zeros((STRIPE, DH), jnp.float32)

    h2, normi = _gcn_kernel(feat3, edges, zrows)
    return _final(feat, h2, normi.reshape(NPAD, 1))


# speedup vs baseline: 1.1290x; 1.1290x over previous
import functools

import jax
import jax.numpy as jnp
from jax import lax
from jax.experimental import pallas as pl
from jax.experimental.pallas import tpu as pltpu
from jax.experimental.pallas import tpu_sc as plsc

N = 10000
E = 320000
D = 128
DH = D // 2
NC, NS = 2, 16
NW = NC * NS
CH = 125
EPT = E // NW
NCH1 = EPT // CH
EPS = E // NS
CH3 = 125
NCH3 = EPS // CH3
NPAD = 10240
STRIPE = NPAD // NS

_mesh = plsc.VectorSubcoreMesh(core_axis_name="c", subcore_axis_name="s")


@functools.partial(
    pl.kernel,
    out_type=jax.ShapeDtypeStruct((NC, NPAD, 16), jnp.float32),
    mesh=_mesh,
    scratch_types=[
        pltpu.VMEM((NCH1, CH), jnp.int32),
        pltpu.VMEM((NCH1, CH), jnp.int32),
        pltpu.VMEM((CH, 16), jnp.float32),
        pltpu.VMEM((CH, 16), jnp.float32),
        pltpu.VMEM_SHARED((NPAD, 16), jnp.float32),
        pltpu.SemaphoreType.DMA,
        pltpu.SemaphoreType.DMA,
    ],
    compiler_params=pltpu.CompilerParams(use_tc_tiling_on_sc=False),
)
def _deg_kernel(edges_hbm, e0_hbm, e1_hbm, z16_hbm, degp_hbm,
                idx_s, idx_d, ev0, ev1, hist_sh, sem0, sem1):
    c = lax.axis_index("c")
    s = lax.axis_index("s")
    w = c * NS + s

    pltpu.sync_copy(z16_hbm, hist_sh.at[pl.ds(s * STRIPE, STRIPE)])
    pltpu.sync_copy(e0_hbm, ev0)
    pltpu.sync_copy(e1_hbm, ev1)
    pltpu.sync_copy(edges_hbm.at[0, w], idx_s)
    pltpu.sync_copy(edges_hbm.at[1, w], idx_d)
    plsc.subcore_barrier()

    W = 4

    def scat(g, src, sem):
        pltpu.async_copy(src, hist_sh.at[idx_s.at[g]], sem, add=True)

    def scat_d(g, src, sem):
        pltpu.async_copy(src, hist_sh.at[idx_d.at[g]], sem, add=True)

    def wait_pair(g):
        pltpu.make_async_copy(ev0, hist_sh.at[idx_s.at[g]], sem0).wait()
        pltpu.make_async_copy(ev1, hist_sh.at[idx_d.at[g]], sem1).wait()

    for g in range(W):
        scat(g, ev0, sem0)
        scat_d(g, ev1, sem1)

    def body(g, carry):
        scat(g, ev0, sem0)
        scat_d(g, ev1, sem1)
        wait_pair(g - W)
        return carry

    lax.fori_loop(W, NCH1, body, 0)
    for g in range(NCH1 - W, NCH1):
        wait_pair(g)
    plsc.subcore_barrier()
    pltpu.sync_copy(hist_sh.at[pl.ds(s * STRIPE, STRIPE)],
                    degp_hbm.at[c, pl.ds(s * STRIPE, STRIPE)])


@functools.partial(
    pl.kernel,
    out_type=jax.ShapeDtypeStruct((NC, NPAD, DH), jnp.float32),
    mesh=_mesh,
    scratch_types=[
        pltpu.VMEM((NCH3, CH3), jnp.int32),
        pltpu.VMEM((NCH3, CH3), jnp.int32),
        pltpu.VMEM((4, CH3, DH), jnp.float32),
        pltpu.VMEM_SHARED((NPAD, DH), jnp.float32),
        [pltpu.SemaphoreType.DMA] * 4,
        [pltpu.SemaphoreType.DMA] * 4,
    ],
    compiler_params=pltpu.CompilerParams(use_tc_tiling_on_sc=False),
)
def _agg_kernel(featsrc_hbm, edges_hbm, zrows_hbm, out_hbm,
                idx_s, idx_d, rows, h_sh, sem_g, sem_s):
    c = lax.axis_index("c")
    s = lax.axis_index("s")

    pltpu.sync_copy(zrows_hbm, h_sh.at[pl.ds(s * STRIPE, STRIPE)])
    pltpu.sync_copy(edges_hbm.at[0, s], idx_s)
    pltpu.sync_copy(edges_hbm.at[1, s], idx_d)
    plsc.subcore_barrier()

    fsrc = featsrc_hbm.at[c]

    def gather(g, b):
        pltpu.async_copy(fsrc.at[idx_s.at[g]], rows.at[b], sem_g[b])

    def wait_gather(g, b):
        pltpu.make_async_copy(fsrc.at[idx_s.at[g]], rows.at[b], sem_g[b]).wait()

    def scatter(g, b):
        pltpu.async_copy(rows.at[b], h_sh.at[idx_d.at[g]], sem_s[b], add=True)

    def wait_scatter(g, b):
        pltpu.make_async_copy(rows.at[b], h_sh.at[idx_d.at[g]],
                              sem_s[b]).wait()

    def step(g, b_main, b_pre, skip_ws=False, skip_gather=False):
        if not skip_ws:
            wait_scatter(g - 2, b_pre)
        if not skip_gather:
            gather(g + 2, b_pre)
        wait_gather(g, b_main)
        scatter(g, b_main)

    gather(0, 0)
    gather(1, 1)
    step(0, 0, 2, skip_ws=True)
    step(1, 1, 3, skip_ws=True)

    def body(gg, carry):
        g = 4 * gg + 2
        for k in range(4):
            step(g + k, (2 + k) % 4, k % 4)
        return carry

    lax.fori_loop(0, (NCH3 - 4) // 4, body, 0)
    step(NCH3 - 2, (NCH3 - 2) % 4, NCH3 % 4, skip_gather=True)
    step(NCH3 - 1, (NCH3 - 1) % 4, (NCH3 + 1) % 4, skip_gather=True)
    wait_scatter(NCH3 - 2, (NCH3 - 2) % 4)
    wait_scatter(NCH3 - 1, (NCH3 - 1) % 4)

    plsc.subcore_barrier()
    pltpu.sync_copy(h_sh.at[pl.ds(s * STRIPE, STRIPE)],
                    out_hbm.at[c, pl.ds(s * STRIPE, STRIPE)])


_RB = 2000


def _scale_body(feat_ref, deg_ref, out_ref):
    d = deg_ref[0, :, 0:1] + deg_ref[1, :, 0:1]
    norm = lax.rsqrt(jnp.maximum(d, 1.0) + 1.0)
    scaled = feat_ref[...] * norm
    out_ref[0] = scaled[:, :DH]
    out_ref[1] = scaled[:, DH:]


def _final_body(feat_ref, h_ref, deg_ref, out_ref):
    d = deg_ref[0, :, 1:2] + deg_ref[1, :, 1:2]
    norm = lax.rsqrt(jnp.maximum(d, 1.0) + 1.0)
    h = jnp.concatenate([h_ref[0], h_ref[1]], axis=1)
    out_ref[...] = (feat_ref[...] + h) * norm


def _scale(feat, degp):
    return pl.pallas_call(
        _scale_body,
        grid=(N // _RB,),
        in_specs=[
            pl.BlockSpec((_RB, D), lambda i: (i, 0)),
            pl.BlockSpec((NC, _RB, 16), lambda i: (0, i, 0)),
        ],
        out_specs=pl.BlockSpec((NC, _RB, DH), lambda i: (0, i, 0)),
        out_shape=jax.ShapeDtypeStruct((NC, N, DH), jnp.float32),
    )(feat, degp)


def _final(feat, h2, degp):
    return pl.pallas_call(
        _final_body,
        grid=(N // _RB,),
        in_specs=[
            pl.BlockSpec((_RB, D), lambda i: (i, 0)),
            pl.BlockSpec((NC, _RB, DH), lambda i: (0, i, 0)),
            pl.BlockSpec((NC, _RB, 16), lambda i: (0, i, 0)),
        ],
        out_specs=pl.BlockSpec((_RB, D), lambda i: (i, 0)),
        out_shape=jax.ShapeDtypeStruct((N, D), jnp.float32),
    )(feat, h2, degp)


def kernel(feat, edge_index):
    ei = edge_index.astype(jnp.int32)
    edges_k1 = ei.reshape(2, NW, NCH1, CH)
    edges_k3 = ei.reshape(2, NS, NCH3, CH3)

    e0 = jnp.zeros((CH, 16), jnp.float32).at[:, 0].set(1.0)
    e1 = jnp.zeros((CH, 16), jnp.float32).at[:, 1].set(1.0)
    z16 = jnp.zeros((STRIPE, 16), jnp.float32)
    zrows = jnp.zeros((STRIPE, DH), jnp.float32)

    degp = _deg_kernel(edges_k1, e0, e1, z16)
    feat_src = _scale(feat, degp)
    h2 = _agg_kernel(feat_src, edges_k3, zrows)
    return _final(feat, h2, degp)


# speedup vs baseline: 1.1625x; 1.0296x over previous
import functools

import jax
import jax.numpy as jnp
from jax import lax
from jax.experimental import pallas as pl
from jax.experimental.pallas import tpu as pltpu
from jax.experimental.pallas import tpu_sc as plsc

N = 10000
E = 320000
D = 128
DH = D // 2
NC, NS = 2, 16
NW = NC * NS
CH = 125
EPT = E // NW
NCH1 = EPT // CH
EPS = E // NS
CH3 = 125
NCH3 = EPS // CH3
NPAD = 10240
STRIPE = NPAD // NS

_mesh = plsc.VectorSubcoreMesh(core_axis_name="c", subcore_axis_name="s")


def _make_deg_kernel(which):
    @functools.partial(
        pl.kernel,
        out_type=jax.ShapeDtypeStruct((NC, NPAD, 16), jnp.float32),
        mesh=_mesh,
        scratch_types=[
            pltpu.VMEM((NCH1, CH), jnp.int32),
            pltpu.VMEM((CH, 16), jnp.float32),
            pltpu.VMEM_SHARED((NPAD, 16), jnp.float32),
            pltpu.SemaphoreType.DMA,
        ],
        compiler_params=pltpu.CompilerParams(use_tc_tiling_on_sc=False),
        name=f"deg_hist_{which}",
    )
    def deg_kernel(edges_hbm, e0_hbm, z16_hbm, degp_hbm,
                   idx, ev0, hist_sh, sem0):
        c = lax.axis_index("c")
        s = lax.axis_index("s")
        w = c * NS + s

        pltpu.sync_copy(z16_hbm, hist_sh.at[pl.ds(s * STRIPE, STRIPE)])
        pltpu.sync_copy(e0_hbm, ev0)
        pltpu.sync_copy(edges_hbm.at[which, w], idx)
        plsc.subcore_barrier()

        W = 4

        def scat(g):
            pltpu.async_copy(ev0, hist_sh.at[idx.at[g]], sem0, add=True)

        def wait_one(g):
            pltpu.make_async_copy(ev0, hist_sh.at[idx.at[g]], sem0).wait()

        for g in range(W):
            scat(g)

        def body(g, carry):
            scat(g)
            wait_one(g - W)
            return carry

        lax.fori_loop(W, NCH1, body, 0)
        for g in range(NCH1 - W, NCH1):
            wait_one(g)
        plsc.subcore_barrier()
        pltpu.sync_copy(hist_sh.at[pl.ds(s * STRIPE, STRIPE)],
                        degp_hbm.at[c, pl.ds(s * STRIPE, STRIPE)])

    return deg_kernel


_deg_out_kernel = _make_deg_kernel(0)
_deg_in_kernel = _make_deg_kernel(1)


@functools.partial(
    pl.kernel,
    out_type=jax.ShapeDtypeStruct((NC, NPAD, DH), jnp.float32),
    mesh=_mesh,
    scratch_types=[
        pltpu.VMEM((NCH3, CH3), jnp.int32),
        pltpu.VMEM((NCH3, CH3), jnp.int32),
        pltpu.VMEM((4, CH3, DH), jnp.float32),
        pltpu.VMEM_SHARED((NPAD, DH), jnp.float32),
        [pltpu.SemaphoreType.DMA] * 4,
        [pltpu.SemaphoreType.DMA] * 4,
    ],
    compiler_params=pltpu.CompilerParams(use_tc_tiling_on_sc=False),
)
def _agg_kernel(featsrc_hbm, edges_hbm, zrows_hbm, out_hbm,
                idx_s, idx_d, rows, h_sh, sem_g, sem_s):
    c = lax.axis_index("c")
    s = lax.axis_index("s")

    pltpu.sync_copy(zrows_hbm, h_sh.at[pl.ds(s * STRIPE, STRIPE)])
    pltpu.sync_copy(edges_hbm.at[0, s], idx_s)
    pltpu.sync_copy(edges_hbm.at[1, s], idx_d)
    plsc.subcore_barrier()

    fsrc = featsrc_hbm.at[c]

    def gather(g, b):
        pltpu.async_copy(fsrc.at[idx_s.at[g]], rows.at[b], sem_g[b])

    def wait_gather(g, b):
        pltpu.make_async_copy(fsrc.at[idx_s.at[g]], rows.at[b], sem_g[b]).wait()

    def scatter(g, b):
        pltpu.async_copy(rows.at[b], h_sh.at[idx_d.at[g]], sem_s[b], add=True)

    def wait_scatter(g, b):
        pltpu.make_async_copy(rows.at[b], h_sh.at[idx_d.at[g]],
                              sem_s[b]).wait()

    def step(g, b_main, b_pre, skip_ws=False, skip_gather=False):
        if not skip_ws:
            wait_scatter(g - 2, b_pre)
        if not skip_gather:
            gather(g + 2, b_pre)
        wait_gather(g, b_main)
        scatter(g, b_main)

    gather(0, 0)
    gather(1, 1)
    step(0, 0, 2, skip_ws=True)
    step(1, 1, 3, skip_ws=True)

    def body(gg, carry):
        g = 4 * gg + 2
        for k in range(4):
            step(g + k, (2 + k) % 4, k % 4)
        return carry

    lax.fori_loop(0, (NCH3 - 4) // 4, body, 0)
    step(NCH3 - 2, (NCH3 - 2) % 4, NCH3 % 4, skip_gather=True)
    step(NCH3 - 1, (NCH3 - 1) % 4, (NCH3 + 1) % 4, skip_gather=True)
    wait_scatter(NCH3 - 2, (NCH3 - 2) % 4)
    wait_scatter(NCH3 - 1, (NCH3 - 1) % 4)

    plsc.subcore_barrier()
    pltpu.sync_copy(h_sh.at[pl.ds(s * STRIPE, STRIPE)],
                    out_hbm.at[c, pl.ds(s * STRIPE, STRIPE)])


_RB = 2000


def _scale_body(feat_ref, deg_ref, out_ref):
    d = deg_ref[0, :, 0:1] + deg_ref[1, :, 0:1]
    norm = lax.rsqrt(jnp.maximum(d, 1.0) + 1.0)
    scaled = feat_ref[...] * norm
    out_ref[0] = scaled[:, :DH]
    out_ref[1] = scaled[:, DH:]


def _final_body(feat_ref, h_ref, deg_ref, out_ref):
    d = deg_ref[0, :, 0:1] + deg_ref[1, :, 0:1]
    norm = lax.rsqrt(jnp.maximum(d, 1.0) + 1.0)
    h = jnp.concatenate([h_ref[0], h_ref[1]], axis=1)
    out_ref[...] = (feat_ref[...] + h) * norm


def _scale(feat, degp):
    return pl.pallas_call(
        _scale_body,
        grid=(N // _RB,),
        in_specs=[
            pl.BlockSpec((_RB, D), lambda i: (i, 0)),
            pl.BlockSpec((NC, _RB, 16), lambda i: (0, i, 0)),
        ],
        out_specs=pl.BlockSpec((NC, _RB, DH), lambda i: (0, i, 0)),
        out_shape=jax.ShapeDtypeStruct((NC, N, DH), jnp.float32),
    )(feat, degp)


def _final(feat, h2, degp):
    return pl.pallas_call(
        _final_body,
        grid=(N // _RB,),
        in_specs=[
            pl.BlockSpec((_RB, D), lambda i: (i, 0)),
            pl.BlockSpec((NC, _RB, DH), lambda i: (0, i, 0)),
            pl.BlockSpec((NC, _RB, 16), lambda i: (0, i, 0)),
        ],
        out_specs=pl.BlockSpec((_RB, D), lambda i: (i, 0)),
        out_shape=jax.ShapeDtypeStruct((N, D), jnp.float32),
    )(feat, h2, degp)


def kernel(feat, edge_index):
    ei = edge_index.astype(jnp.int32)
    edges_k1 = ei.reshape(2, NW, NCH1, CH)
    edges_k3 = ei.reshape(2, NS, NCH3, CH3)

    e0 = jnp.zeros((CH, 16), jnp.float32).at[:, 0].set(1.0)
    z16 = jnp.zeros((STRIPE, 16), jnp.float32)
    zrows = jnp.zeros((STRIPE, DH), jnp.float32)

    dego = _deg_out_kernel(edges_k1, e0, z16)
    degi = _deg_in_kernel(edges_k1, e0, z16)
    feat_src = _scale(feat, dego)
    h2 = _agg_kernel(feat_src, edges_k3, zrows)
    return _final(feat, h2, degi)
